# baseline (speedup 1.0000x reference)
import functools

import jax
import jax.numpy as jnp
from jax.experimental import pallas as pl
from jax.experimental.pallas import tpu as pltpu

N_EXPERTS = 64
K = 8
ALPHA = 0.01


def _route_block(x, w, ps_acc, cnt_acc):
    lt = jax.lax.dot_general(w, x, (((1,), (1,)), ((), ())),
                             preferred_element_type=jnp.float32)
    m = jnp.max(lt, axis=0, keepdims=True)
    ex = jnp.exp(lt - m)
    s = jnp.sum(ex, axis=0, keepdims=True)
    p = ex / s

    ps_acc[...] += p

    chunk = 512
    b = p.shape[1]
    tw_parts = []
    ti_parts = []
    cnt = []
    for c in range(b // chunk):
        work = p[:, c * chunk:(c + 1) * chunk]
        iota = jax.lax.broadcasted_iota(jnp.int32, work.shape, 0)
        ws = []
        idxs = []
        for _ in range(K):
            mx = jnp.max(work, axis=0, keepdims=True)
            sel = jnp.min(jnp.where(work == mx, iota, N_EXPERTS),
                          axis=0, keepdims=True)
            work = jnp.where(iota == sel, -1.0, work)
            ws.append(mx)
            idxs.append(sel)
        cnt.append((work < 0).astype(jnp.float32))
        tw_parts.append(jnp.concatenate(ws, axis=0))
        ti_parts.append(jnp.concatenate(idxs, axis=0))
    cnt_acc[...] += jnp.concatenate(cnt, axis=1)

    return jnp.concatenate(tw_parts, axis=1), jnp.concatenate(ti_parts, axis=1)


def _gate_kernel(hs1_ref, hs2_ref, w_ref, tw_ref, ti_ref, aux_ref,
                 ps_acc, cnt_acc, *, num_steps, total_tokens, block):
    step = pl.program_id(0)

    @pl.when(step == 0)
    def _init():
        ps_acc[...] = jnp.zeros_like(ps_acc)
        cnt_acc[...] = jnp.zeros_like(cnt_acc)

    w = w_ref[...]
    tw1, ti1 = _route_block(hs1_ref[...], w, ps_acc, cnt_acc)
    tw2, ti2 = _route_block(hs2_ref[...], w, ps_acc, cnt_acc)
    tw_ref[:, 0:block] = tw1
    tw_ref[:, block:2 * block] = tw2
    ti_ref[:, 0:block] = ti1
    ti_ref[:, block:2 * block] = ti2

    @pl.when(step == num_steps - 1)
    def _finish():
        pi = jnp.sum(ps_acc[...], axis=1)
        c = jnp.sum(cnt_acc[...], axis=1)
        scale = ALPHA * N_EXPERTS / (float(total_tokens) * float(total_tokens) * K)
        aux_ref[0, 0] = jnp.sum(pi * c) * scale


@jax.jit
def _gate(hs, w):
    t, h = hs.shape
    block = 4096
    num_steps = t // (2 * block)
    kfn = functools.partial(_gate_kernel, num_steps=num_steps,
                            total_tokens=t, block=block)
    tw, ti, aux = pl.pallas_call(
        kfn,
        grid=(num_steps,),
        in_specs=[
            pl.BlockSpec((block, h), lambda i: (2 * i, 0)),
            pl.BlockSpec((block, h), lambda i: (2 * i + 1, 0)),
            pl.BlockSpec((N_EXPERTS, h), lambda i: (0, 0)),
        ],
        out_specs=[
            pl.BlockSpec((K, 2 * block), lambda i: (0, i)),
            pl.BlockSpec((K, 2 * block), lambda i: (0, i)),
            pl.BlockSpec(memory_space=pltpu.SMEM),
        ],
        out_shape=[
            jax.ShapeDtypeStruct((K, t), jnp.float32),
            jax.ShapeDtypeStruct((K, t), jnp.int32),
            jax.ShapeDtypeStruct((1, 1), jnp.float32),
        ],
        scratch_shapes=[
            pltpu.VMEM((N_EXPERTS, block), jnp.float32),
            pltpu.VMEM((N_EXPERTS, block), jnp.float32),
        ],
        compiler_params=pltpu.CompilerParams(
            dimension_semantics=("arbitrary",),
            vmem_limit_bytes=100 * 1024 * 1024,
        ),
    )(hs, hs, w)
    return tw.T, ti.T, aux[0, 0]


def kernel(hidden_states, weight):
    bsz, seq_len, h = hidden_states.shape
    hs = hidden_states.reshape(-1, h)
    tw, ti, aux = _gate(hs, weight)
    return tw, ti, aux


# speedup vs baseline: 1.0369x; 1.0369x over previous
import functools

import jax
import jax.numpy as jnp
from jax.experimental import pallas as pl
from jax.experimental.pallas import tpu as pltpu

N_EXPERTS = 64
K = 8
ALPHA = 0.01


def _route_block(x, w, ps_acc, cnt_acc):
    lt = jax.lax.dot_general(w, x, (((1,), (1,)), ((), ())),
                             preferred_element_type=jnp.float32)
    m = jnp.max(lt, axis=0, keepdims=True)
    ex = jnp.exp(lt - m)
    s = jnp.sum(ex, axis=0, keepdims=True)
    p = ex / s

    ps_acc[...] += p

    chunk = 512
    b = p.shape[1]
    tw_parts = []
    ti_parts = []
    cnt = []
    for c in range(b // chunk):
        work = p[:, c * chunk:(c + 1) * chunk]
        iota = jax.lax.broadcasted_iota(jnp.int32, work.shape, 0)
        ws = []
        idxs = []
        for _ in range(K):
            mx = jnp.max(work, axis=0, keepdims=True)
            sel = jnp.min(jnp.where(work == mx, iota, N_EXPERTS),
                          axis=0, keepdims=True)
            work = jnp.where(iota == sel, -1.0, work)
            ws.append(mx)
            idxs.append(sel)
        cnt.append((work < 0).astype(jnp.float32))
        tw_parts.append(jnp.concatenate(ws, axis=0))
        ti_parts.append(jnp.concatenate(idxs, axis=0))
    cnt_acc[...] += jnp.concatenate(cnt, axis=1)

    return jnp.concatenate(tw_parts, axis=1), jnp.concatenate(ti_parts, axis=1)


def _gate_kernel(hs1_ref, hs2_ref, w_ref, tw_ref, ti_ref, aux_ref,
                 ps_acc, cnt_acc, *, num_steps, total_tokens, block):
    step = pl.program_id(0)

    @pl.when(step == 0)
    def _init():
        ps_acc[...] = jnp.zeros_like(ps_acc)
        cnt_acc[...] = jnp.zeros_like(cnt_acc)

    w = w_ref[...]
    tw1, ti1 = _route_block(hs1_ref[...], w, ps_acc, cnt_acc)
    tw2, ti2 = _route_block(hs2_ref[...], w, ps_acc, cnt_acc)
    tw_ref[:, 0:block] = tw1
    tw_ref[:, block:2 * block] = tw2
    ti_ref[:, 0:block] = ti1
    ti_ref[:, block:2 * block] = ti2

    @pl.when(step == num_steps - 1)
    def _finish():
        pi = jnp.sum(ps_acc[...], axis=1)
        c = jnp.sum(cnt_acc[...], axis=1)
        scale = ALPHA * N_EXPERTS / (float(total_tokens) * float(total_tokens) * K)
        aux_ref[0, 0] = jnp.sum(pi * c) * scale


@jax.jit
def _gate(hs, w):
    t, h = hs.shape
    block = 2048
    num_steps = t // (2 * block)
    kfn = functools.partial(_gate_kernel, num_steps=num_steps,
                            total_tokens=t, block=block)
    tw, ti, aux = pl.pallas_call(
        kfn,
        grid=(num_steps,),
        in_specs=[
            pl.BlockSpec((block, h), lambda i: (2 * i, 0)),
            pl.BlockSpec((block, h), lambda i: (2 * i + 1, 0)),
            pl.BlockSpec((N_EXPERTS, h), lambda i: (0, 0)),
        ],
        out_specs=[
            pl.BlockSpec((K, 2 * block), lambda i: (0, i)),
            pl.BlockSpec((K, 2 * block), lambda i: (0, i)),
            pl.BlockSpec(memory_space=pltpu.SMEM),
        ],
        out_shape=[
            jax.ShapeDtypeStruct((K, t), jnp.float32),
            jax.ShapeDtypeStruct((K, t), jnp.int32),
            jax.ShapeDtypeStruct((1, 1), jnp.float32),
        ],
        scratch_shapes=[
            pltpu.VMEM((N_EXPERTS, block), jnp.float32),
            pltpu.VMEM((N_EXPERTS, block), jnp.float32),
        ],
        compiler_params=pltpu.CompilerParams(
            dimension_semantics=("arbitrary",),
            vmem_limit_bytes=100 * 1024 * 1024,
        ),
    )(hs, hs, w)
    return tw.T, ti.T, aux[0, 0]


def kernel(hidden_states, weight):
    bsz, seq_len, h = hidden_states.shape
    hs = hidden_states.reshape(-1, h)
    tw, ti, aux = _gate(hs, weight)
    return tw, ti, aux
